# baseline (speedup 1.0000x reference)
import functools

import jax
import jax.numpy as jnp
from jax import lax
from jax.experimental import pallas as pl
from jax.experimental.pallas import tpu as pltpu
from jax.experimental.pallas import tpu_sc as plsc

N = 10000
NP = 10112
NPD = 10240
E = 320000
EPAD = 327680
D = 128
NC = 2
NS = 16
NW = NC * NS
B = 64
KB = EPAD // (NW * B)
SEG = 10
NSEG = KB // SEG
STRIPE = NP // NS
DSTRIPE = NPD // NS

_mesh = plsc.VectorSubcoreMesh(core_axis_name="c", subcore_axis_name="s")


def _sc_agg_body(with_deg):
    def body(*args):
        if with_deg:
            (g_hbm, srcr_hbm, dstr_hbm, z2d_hbm, z1d_hbm, ones_hbm,
             part_hbm, deg0_hbm, deg1_hbm,
             srcseg, dstseg, rows_v, ones_v, acc_sh, deg_sh,
             g0, g1, g2, g3, g4, s0, s1, s2, s3, s4, t0, t1, dsem) = args
        else:
            (g_hbm, srcr_hbm, dstr_hbm, z2d_hbm,
             part_hbm,
             srcseg, dstseg, rows_v, acc_sh,
             g0, g1, g2, g3, g4, s0, s1, s2, s3, s4, t0, t1) = args
        gsems = (g0, g1, g2, g3, g4)
        ssems = (s0, s1, s2, s3, s4)
        stsems = (t0, t1)
        c = lax.axis_index("c")
        s = lax.axis_index("s")
        w = c * NS + s
        pltpu.sync_copy(srcr_hbm.at[w, 0], srcseg.at[0])
        pltpu.sync_copy(dstr_hbm.at[w, 0], dstseg.at[0])
        if with_deg:
            pltpu.sync_copy(ones_hbm, ones_v)
            pltpu.sync_copy(z1d_hbm.at[pl.ds(s * DSTRIPE, DSTRIPE)],
                            deg_sh.at[pl.ds(s * DSTRIPE, DSTRIPE)])
        pltpu.sync_copy(z2d_hbm.at[pl.ds(s * STRIPE, STRIPE)],
                        acc_sh.at[pl.ds(s * STRIPE, STRIPE)])
        plsc.subcore_barrier()

        def fire_stage(t1_, slot):
            pltpu.async_copy(srcr_hbm.at[w, t1_], srcseg.at[slot],
                             stsems[slot])
            pltpu.async_copy(dstr_hbm.at[w, t1_], dstseg.at[slot],
                             stsems[slot])

        def wait_stage(t1_, slot):
            pltpu.make_async_copy(srcr_hbm.at[w, t1_], srcseg.at[slot],
                                  stsems[slot]).wait()
            pltpu.make_async_copy(dstr_hbm.at[w, t1_], dstseg.at[slot],
                                  stsems[slot]).wait()

        def run_seg(p):
            sseg = srcseg.at[p]
            dseg = dstseg.at[p]

            def fire_g(r, j):
                pltpu.async_copy(g_hbm.at[sseg.at[r]], rows_v.at[j], gsems[j])

            def wait_g(r, j):
                pltpu.make_async_copy(g_hbm.at[sseg.at[r]],
                                      rows_v.at[j], gsems[j]).wait()

            def fire_s(r, j):
                pltpu.async_copy(rows_v.at[j], acc_sh.at[dseg.at[r]],
                                 ssems[j], add=True)
                if with_deg:
                    pltpu.async_copy(ones_v, deg_sh.at[dseg.at[r]],
                                     dsem, add=True)

            def wait_s(r, j):
                pltpu.make_async_copy(rows_v.at[j], acc_sh.at[dseg.at[r]],
                                      ssems[j]).wait()

            fire_g(0, 0)
            fire_g(1, 1)
            fire_g(2, 2)

            def rr_body(rr, carry):
                base = 5 * rr
                for i in range(5):
                    r = base + i
                    j3 = (i + 3) % 5
                    if i < 2:
                        @pl.when(rr > 0)
                        def _(j3=j3, r=r):
                            wait_s(r - 2, j3)

                        fire_g(r + 3, j3)
                    else:
                        @pl.when(rr == 0)
                        def _(j3=j3, r=r):
                            wait_s(r - 2, j3)
                            fire_g(r + 3, j3)

                    wait_g(r, i)
                    fire_s(r, i)
                return carry

            lax.fori_loop(0, SEG // 5, rr_body, 0)
            for i in range(5):
                wait_s(SEG - 5 + i, i)

        def t_body(t, carry):
            def go(p):
                @pl.when(t < NSEG - 1)
                def _():
                    fire_stage(t + 1, 1 - p)

                run_seg(p)

                @pl.when(t < NSEG - 1)
                def _():
                    wait_stage(t + 1, 1 - p)

            @pl.when(lax.rem(t, 2) == 0)
            def _():
                go(0)

            @pl.when(lax.rem(t, 2) == 1)
            def _():
                go(1)

            return carry

        lax.fori_loop(0, NSEG, t_body, 0)

        if with_deg:
            def d_body(k, carry):
                pltpu.make_async_copy(ones_v, deg_sh.at[dstseg.at[0, 0]],
                                      dsem).wait()
                return carry
            lax.fori_loop(0, KB, d_body, 0)

        plsc.subcore_barrier()
        sl = pl.ds(s * STRIPE, STRIPE)
        pltpu.sync_copy(acc_sh.at[sl], part_hbm.at[c, sl])
        if with_deg:
            dsl = pl.ds(s * DSTRIPE, DSTRIPE)

            @pl.when(c == 0)
            def _():
                pltpu.sync_copy(deg_sh.at[dsl], deg0_hbm.at[dsl])

            @pl.when(c == 1)
            def _():
                pltpu.sync_copy(deg_sh.at[dsl], deg1_hbm.at[dsl])

    return body


_sc_params = pltpu.CompilerParams(use_tc_tiling_on_sc=False)

_agg_sems = (
    pltpu.SemaphoreType.DMA,
    pltpu.SemaphoreType.DMA,
    pltpu.SemaphoreType.DMA,
    pltpu.SemaphoreType.DMA,
    pltpu.SemaphoreType.DMA,
    pltpu.SemaphoreType.DMA,
    pltpu.SemaphoreType.DMA,
    pltpu.SemaphoreType.DMA,
    pltpu.SemaphoreType.DMA,
    pltpu.SemaphoreType.DMA,
    pltpu.SemaphoreType.DMA,
    pltpu.SemaphoreType.DMA,
)

_sc_agg_deg = functools.partial(
    pl.kernel,
    mesh=_mesh,
    compiler_params=_sc_params,
    out_type=(
        jax.ShapeDtypeStruct((NC, NP, D), jnp.float32),
        jax.ShapeDtypeStruct((NPD,), jnp.float32),
        jax.ShapeDtypeStruct((NPD,), jnp.float32),
    ),
    scratch_types=(
        pltpu.VMEM((2, SEG, B), jnp.int32),
        pltpu.VMEM((2, SEG, B), jnp.int32),
        pltpu.VMEM((5, B, D), jnp.float32),
        pltpu.VMEM((B,), jnp.float32),
        pltpu.VMEM_SHARED((NP, D), jnp.float32),
        pltpu.VMEM_SHARED((NPD,), jnp.float32),
    ) + _agg_sems + (pltpu.SemaphoreType.DMA,),
)(_sc_agg_body(True))


_sc_agg = functools.partial(
    pl.kernel,
    mesh=_mesh,
    compiler_params=_sc_params,
    out_type=jax.ShapeDtypeStruct((NC, NP, D), jnp.float32),
    scratch_types=(
        pltpu.VMEM((2, SEG, B), jnp.int32),
        pltpu.VMEM((2, SEG, B), jnp.int32),
        pltpu.VMEM((5, B, D), jnp.float32),
        pltpu.VMEM_SHARED((NP, D), jnp.float32),
    ) + _agg_sems,
)(_sc_agg_body(False))


RB = 1000


def _mm_body(x_ref, w_ref, o_ref):
    o_ref[...] = jnp.dot(x_ref[...], w_ref[...],
                         preferred_element_type=jnp.float32)


def _tc_matmul(x, W):
    return pl.pallas_call(
        _mm_body,
        grid=(N // RB,),
        in_specs=[pl.BlockSpec((RB, D), lambda i: (i, 0)),
                  pl.BlockSpec((D, D), lambda i: (0, 0))],
        out_specs=pl.BlockSpec((RB, D), lambda i: (i, 0)),
        out_shape=jax.ShapeDtypeStruct((N, D), jnp.float32),
    )(x, W)


def _fused_body(g_ref, p0_ref, p1_ref, d0_ref, d1_ref, b_ref, w_ref, o_ref):
    inv = 1.0 / jnp.maximum(d0_ref[...] + d1_ref[...], 1.0)
    h = g_ref[...] + b_ref[...] + (p0_ref[0] + p1_ref[0]) * inv
    h = jnp.maximum(h, 0.0)
    o_ref[...] = jnp.dot(h, w_ref[...], preferred_element_type=jnp.float32)


def _tc_fused(g, p, d0, d1, b, Wn):
    return pl.pallas_call(
        _fused_body,
        grid=(N // RB,),
        in_specs=[pl.BlockSpec((RB, D), lambda i: (i, 0)),
                  pl.BlockSpec((1, RB, D), lambda i: (0, i, 0)),
                  pl.BlockSpec((1, RB, D), lambda i: (1, i, 0)),
                  pl.BlockSpec((RB, 1), lambda i: (i, 0)),
                  pl.BlockSpec((RB, 1), lambda i: (i, 0)),
                  pl.BlockSpec((1, D), lambda i: (0, 0)),
                  pl.BlockSpec((D, D), lambda i: (0, 0))],
        out_specs=pl.BlockSpec((RB, D), lambda i: (i, 0)),
        out_shape=jax.ShapeDtypeStruct((N, D), jnp.float32),
    )(g, p, p, d0, d1, b, Wn)


def _final_body(g_ref, p0_ref, p1_ref, d0_ref, d1_ref, b_ref, o_ref):
    inv = 1.0 / jnp.maximum(d0_ref[...] + d1_ref[...], 1.0)
    o_ref[...] = (g_ref[...] + b_ref[...]
                  + (p0_ref[0] + p1_ref[0]) * inv)


def _tc_final(g, p, d0, d1, b):
    return pl.pallas_call(
        _final_body,
        grid=(N // RB,),
        in_specs=[pl.BlockSpec((RB, D), lambda i: (i, 0)),
                  pl.BlockSpec((1, RB, D), lambda i: (0, i, 0)),
                  pl.BlockSpec((1, RB, D), lambda i: (1, i, 0)),
                  pl.BlockSpec((RB, 1), lambda i: (i, 0)),
                  pl.BlockSpec((RB, 1), lambda i: (i, 0)),
                  pl.BlockSpec((1, D), lambda i: (0, 0))],
        out_specs=pl.BlockSpec((RB, D), lambda i: (i, 0)),
        out_shape=jax.ShapeDtypeStruct((N, D), jnp.float32),
    )(g, p, p, d0, d1, b)


def kernel(x, edge_index, W1, b1, W2, b2, W3, b3):
    npad = EPAD - E
    srcf = edge_index[0].astype(jnp.int32)
    dstf = edge_index[1].astype(jnp.int32)
    ar = jnp.arange(npad, dtype=jnp.int32)
    pad_src = (ar * 131) % N
    pad_dst = N + (ar % (NP - N))
    srcp = jnp.concatenate([srcf, pad_src]).reshape(NW, NSEG, SEG, B)
    dstp = jnp.concatenate([dstf, pad_dst]).reshape(NW, NSEG, SEG, B)
    z2d = jnp.zeros((NP, D), jnp.float32)
    z1d = jnp.zeros((NPD,), jnp.float32)
    ones = jnp.ones((B,), jnp.float32)
    b1r = b1.reshape(1, D)
    b2r = b2.reshape(1, D)
    b3r = b3.reshape(1, D)

    g1 = _tc_matmul(x, W1)
    p1, dg0, dg1 = _sc_agg_deg(g1, srcp, dstp, z2d, z1d, ones)
    d0 = dg0.reshape(NPD, 1)
    d1 = dg1.reshape(NPD, 1)
    g2 = _tc_fused(g1, p1, d0, d1, b1r, W2)
    p2 = _sc_agg(g2, srcp, dstp, z2d)
    g3 = _tc_fused(g2, p2, d0, d1, b2r, W3)
    p3 = _sc_agg(g3, srcp, dstp, z2d)
    return _tc_final(g3, p3, d0, d1, b3r)


# speedup vs baseline: 1.0083x; 1.0083x over previous
import functools

import jax
import jax.numpy as jnp
from jax import lax
from jax.experimental import pallas as pl
from jax.experimental.pallas import tpu as pltpu
from jax.experimental.pallas import tpu_sc as plsc

N = 10000
NP = 10112
NPD = 10240
E = 320000
EPAD = 327680
D = 128
NC = 2
NS = 16
NW = NC * NS
B = 80
KB = EPAD // (NW * B)
SEG = 16
NSEG = KB // SEG
STRIPE = NP // NS
DSTRIPE = NPD // NS

_mesh = plsc.VectorSubcoreMesh(core_axis_name="c", subcore_axis_name="s")


def _sc_agg_body(with_deg):
    def body(*args):
        if with_deg:
            (g_hbm, srcr_hbm, dstr_hbm, z2d_hbm, z1d_hbm, ones_hbm,
             part_hbm, deg0_hbm, deg1_hbm,
             srcseg, dstseg, rows_v, ones_v, acc_sh, deg_sh,
             g0, g1, g2, g3, s0, s1, s2, s3, t0, t1, dsem) = args
        else:
            (g_hbm, srcr_hbm, dstr_hbm, z2d_hbm,
             part_hbm,
             srcseg, dstseg, rows_v, acc_sh,
             g0, g1, g2, g3, s0, s1, s2, s3, t0, t1) = args
        gsems = (g0, g1, g2, g3)
        ssems = (s0, s1, s2, s3)
        stsems = (t0, t1)
        c = lax.axis_index("c")
        s = lax.axis_index("s")
        w = c * NS + s
        pltpu.sync_copy(srcr_hbm.at[w, 0], srcseg.at[0])
        pltpu.sync_copy(dstr_hbm.at[w, 0], dstseg.at[0])
        if with_deg:
            pltpu.sync_copy(ones_hbm, ones_v)
            pltpu.sync_copy(z1d_hbm.at[pl.ds(s * DSTRIPE, DSTRIPE)],
                            deg_sh.at[pl.ds(s * DSTRIPE, DSTRIPE)])
        pltpu.sync_copy(z2d_hbm.at[pl.ds(s * STRIPE, STRIPE)],
                        acc_sh.at[pl.ds(s * STRIPE, STRIPE)])
        plsc.subcore_barrier()

        def fire_stage(t1_, slot):
            pltpu.async_copy(srcr_hbm.at[w, t1_], srcseg.at[slot],
                             stsems[slot])
            pltpu.async_copy(dstr_hbm.at[w, t1_], dstseg.at[slot],
                             stsems[slot])

        def wait_stage(t1_, slot):
            pltpu.make_async_copy(srcr_hbm.at[w, t1_], srcseg.at[slot],
                                  stsems[slot]).wait()
            pltpu.make_async_copy(dstr_hbm.at[w, t1_], dstseg.at[slot],
                                  stsems[slot]).wait()

        def run_seg(p):
            sseg = srcseg.at[p]
            dseg = dstseg.at[p]

            def fire_g(r, j):
                pltpu.async_copy(g_hbm.at[sseg.at[r]], rows_v.at[j], gsems[j])

            def wait_g(r, j):
                pltpu.make_async_copy(g_hbm.at[sseg.at[r]],
                                      rows_v.at[j], gsems[j]).wait()

            def fire_s(r, j):
                pltpu.async_copy(rows_v.at[j], acc_sh.at[dseg.at[r]],
                                 ssems[j], add=True)
                if with_deg:
                    pltpu.async_copy(ones_v, deg_sh.at[dseg.at[r]],
                                     dsem, add=True)

            def wait_s(r, j):
                pltpu.make_async_copy(rows_v.at[j], acc_sh.at[dseg.at[r]],
                                      ssems[j]).wait()

            fire_g(0, 0)
            fire_g(1, 1)
            fire_g(2, 2)

            def rr_body(rr, carry):
                base = 4 * rr

                @pl.when(rr > 0)
                def _():
                    wait_s(base - 1, 3)

                fire_g(base + 3, 3)
                wait_g(base, 0)
                fire_s(base, 0)
                wait_g(base + 1, 1)
                fire_s(base + 1, 1)
                wait_s(base, 0)

                @pl.when(rr < SEG // 4 - 1)
                def _():
                    fire_g(base + 4, 0)

                wait_g(base + 2, 2)
                fire_s(base + 2, 2)
                wait_s(base + 1, 1)

                @pl.when(rr < SEG // 4 - 1)
                def _():
                    fire_g(base + 5, 1)

                wait_g(base + 3, 3)
                fire_s(base + 3, 3)
                wait_s(base + 2, 2)

                @pl.when(rr < SEG // 4 - 1)
                def _():
                    fire_g(base + 6, 2)

                return carry

            lax.fori_loop(0, SEG // 4, rr_body, 0)
            wait_s(SEG - 1, 3)

        def t_body(t, carry):
            def go(p):
                @pl.when(t < NSEG - 1)
                def _():
                    fire_stage(t + 1, 1 - p)

                run_seg(p)

                @pl.when(t < NSEG - 1)
                def _():
                    wait_stage(t + 1, 1 - p)

            @pl.when(lax.rem(t, 2) == 0)
            def _():
                go(0)

            @pl.when(lax.rem(t, 2) == 1)
            def _():
                go(1)

            return carry

        lax.fori_loop(0, NSEG, t_body, 0)

        if with_deg:
            def d_body(k, carry):
                pltpu.make_async_copy(ones_v, deg_sh.at[dstseg.at[0, 0]],
                                      dsem).wait()
                return carry
            lax.fori_loop(0, KB, d_body, 0)

        plsc.subcore_barrier()
        sl = pl.ds(s * STRIPE, STRIPE)
        pltpu.sync_copy(acc_sh.at[sl], part_hbm.at[c, sl])
        if with_deg:
            dsl = pl.ds(s * DSTRIPE, DSTRIPE)

            @pl.when(c == 0)
            def _():
                pltpu.sync_copy(deg_sh.at[dsl], deg0_hbm.at[dsl])

            @pl.when(c == 1)
            def _():
                pltpu.sync_copy(deg_sh.at[dsl], deg1_hbm.at[dsl])

    return body


_sc_params = pltpu.CompilerParams(use_tc_tiling_on_sc=False)

_agg_sems = (
    pltpu.SemaphoreType.DMA,
    pltpu.SemaphoreType.DMA,
    pltpu.SemaphoreType.DMA,
    pltpu.SemaphoreType.DMA,
    pltpu.SemaphoreType.DMA,
    pltpu.SemaphoreType.DMA,
    pltpu.SemaphoreType.DMA,
    pltpu.SemaphoreType.DMA,
    pltpu.SemaphoreType.DMA,
    pltpu.SemaphoreType.DMA,
)

_sc_agg_deg = functools.partial(
    pl.kernel,
    mesh=_mesh,
    compiler_params=_sc_params,
    out_type=(
        jax.ShapeDtypeStruct((NC, NP, D), jnp.float32),
        jax.ShapeDtypeStruct((NPD,), jnp.float32),
        jax.ShapeDtypeStruct((NPD,), jnp.float32),
    ),
    scratch_types=(
        pltpu.VMEM((2, SEG, B), jnp.int32),
        pltpu.VMEM((2, SEG, B), jnp.int32),
        pltpu.VMEM((4, B, D), jnp.float32),
        pltpu.VMEM((B,), jnp.float32),
        pltpu.VMEM_SHARED((NP, D), jnp.float32),
        pltpu.VMEM_SHARED((NPD,), jnp.float32),
    ) + _agg_sems + (pltpu.SemaphoreType.DMA,),
)(_sc_agg_body(True))


_sc_agg = functools.partial(
    pl.kernel,
    mesh=_mesh,
    compiler_params=_sc_params,
    out_type=jax.ShapeDtypeStruct((NC, NP, D), jnp.float32),
    scratch_types=(
        pltpu.VMEM((2, SEG, B), jnp.int32),
        pltpu.VMEM((2, SEG, B), jnp.int32),
        pltpu.VMEM((4, B, D), jnp.float32),
        pltpu.VMEM_SHARED((NP, D), jnp.float32),
    ) + _agg_sems,
)(_sc_agg_body(False))


RB = 1000


def _mm_body(x_ref, w_ref, o_ref):
    o_ref[...] = jnp.dot(x_ref[...], w_ref[...],
                         preferred_element_type=jnp.float32)


def _tc_matmul(x, W):
    return pl.pallas_call(
        _mm_body,
        grid=(N // RB,),
        in_specs=[pl.BlockSpec((RB, D), lambda i: (i, 0)),
                  pl.BlockSpec((D, D), lambda i: (0, 0))],
        out_specs=pl.BlockSpec((RB, D), lambda i: (i, 0)),
        out_shape=jax.ShapeDtypeStruct((N, D), jnp.float32),
    )(x, W)


def _fused_body(g_ref, p0_ref, p1_ref, d0_ref, d1_ref, b_ref, w_ref, o_ref):
    inv = 1.0 / jnp.maximum(d0_ref[...] + d1_ref[...], 1.0)
    h = g_ref[...] + b_ref[...] + (p0_ref[0] + p1_ref[0]) * inv
    h = jnp.maximum(h, 0.0)
    o_ref[...] = jnp.dot(h, w_ref[...], preferred_element_type=jnp.float32)


def _tc_fused(g, p, d0, d1, b, Wn):
    return pl.pallas_call(
        _fused_body,
        grid=(N // RB,),
        in_specs=[pl.BlockSpec((RB, D), lambda i: (i, 0)),
                  pl.BlockSpec((1, RB, D), lambda i: (0, i, 0)),
                  pl.BlockSpec((1, RB, D), lambda i: (1, i, 0)),
                  pl.BlockSpec((RB, 1), lambda i: (i, 0)),
                  pl.BlockSpec((RB, 1), lambda i: (i, 0)),
                  pl.BlockSpec((1, D), lambda i: (0, 0)),
                  pl.BlockSpec((D, D), lambda i: (0, 0))],
        out_specs=pl.BlockSpec((RB, D), lambda i: (i, 0)),
        out_shape=jax.ShapeDtypeStruct((N, D), jnp.float32),
    )(g, p, p, d0, d1, b, Wn)


def _final_body(g_ref, p0_ref, p1_ref, d0_ref, d1_ref, b_ref, o_ref):
    inv = 1.0 / jnp.maximum(d0_ref[...] + d1_ref[...], 1.0)
    o_ref[...] = (g_ref[...] + b_ref[...]
                  + (p0_ref[0] + p1_ref[0]) * inv)


def _tc_final(g, p, d0, d1, b):
    return pl.pallas_call(
        _final_body,
        grid=(N // RB,),
        in_specs=[pl.BlockSpec((RB, D), lambda i: (i, 0)),
                  pl.BlockSpec((1, RB, D), lambda i: (0, i, 0)),
                  pl.BlockSpec((1, RB, D), lambda i: (1, i, 0)),
                  pl.BlockSpec((RB, 1), lambda i: (i, 0)),
                  pl.BlockSpec((RB, 1), lambda i: (i, 0)),
                  pl.BlockSpec((1, D), lambda i: (0, 0))],
        out_specs=pl.BlockSpec((RB, D), lambda i: (i, 0)),
        out_shape=jax.ShapeDtypeStruct((N, D), jnp.float32),
    )(g, p, p, d0, d1, b)


def kernel(x, edge_index, W1, b1, W2, b2, W3, b3):
    npad = EPAD - E
    srcf = edge_index[0].astype(jnp.int32)
    dstf = edge_index[1].astype(jnp.int32)
    ar = jnp.arange(npad, dtype=jnp.int32)
    pad_src = (ar * 131) % N
    pad_dst = N + (ar % (NP - N))
    srcp = jnp.concatenate([srcf, pad_src]).reshape(NW, NSEG, SEG, B)
    dstp = jnp.concatenate([dstf, pad_dst]).reshape(NW, NSEG, SEG, B)
    z2d = jnp.zeros((NP, D), jnp.float32)
    z1d = jnp.zeros((NPD,), jnp.float32)
    ones = jnp.ones((B,), jnp.float32)
    b1r = b1.reshape(1, D)
    b2r = b2.reshape(1, D)
    b3r = b3.reshape(1, D)

    g1 = _tc_matmul(x, W1)
    p1, dg0, dg1 = _sc_agg_deg(g1, srcp, dstp, z2d, z1d, ones)
    d0 = dg0.reshape(NPD, 1)
    d1 = dg1.reshape(NPD, 1)
    g2 = _tc_fused(g1, p1, d0, d1, b1r, W2)
    p2 = _sc_agg(g2, srcp, dstp, z2d)
    g3 = _tc_fused(g2, p2, d0, d1, b2r, W3)
    p3 = _sc_agg(g3, srcp, dstp, z2d)
    return _tc_final(g3, p3, d0, d1, b3r)
